# baseline (speedup 1.0000x reference)
import functools

import jax
import jax.numpy as jnp
from jax import lax
from jax.experimental import pallas as pl
from jax.experimental.pallas import tpu as pltpu
from jax.experimental.pallas import tpu_sc as plsc

N, D, E = 8192, 4096, 64
ROWS = 1024
NW = 32
RW = N // NW

_LN2 = 0.6931471805599453
_C5, _C4, _C3, _C2, _C1, _C0 = (0.030104854, -0.130124481, 0.283308377,
                                -0.489158198, 0.999010615, 2.2112210e-05)


def _logits_body(x_ref, w_ref, b_ref, out_ref):
    out_ref[...] = jnp.dot(x_ref[...], w_ref[...],
                           preferred_element_type=jnp.float32) + b_ref[...]


def _tc_logits(x, W, b2):
    grid_spec = pl.GridSpec(
        grid=(N // ROWS,),
        in_specs=[
            pl.BlockSpec((ROWS, D), lambda i: (i, 0)),
            pl.BlockSpec((D, E), lambda i: (0, 0)),
            pl.BlockSpec((1, E), lambda i: (0, 0)),
        ],
        out_specs=[pl.BlockSpec((ROWS, E), lambda i: (i, 0))],
    )
    return pl.pallas_call(
        _logits_body, grid_spec=grid_spec,
        out_shape=[jax.ShapeDtypeStruct((N, E), jnp.float32)])(x, W, b2)[0]


_sc_mesh = plsc.VectorSubcoreMesh(core_axis_name="c", subcore_axis_name="s")


@functools.partial(
    pl.kernel,
    out_type=(
        jax.ShapeDtypeStruct((N, E), jnp.float32),
        jax.ShapeDtypeStruct((NW, 16, 16), jnp.int32),
        jax.ShapeDtypeStruct((NW, 16, 16), jnp.float32),
        jax.ShapeDtypeStruct((NW, E), jnp.float32),
        jax.ShapeDtypeStruct((NW, E), jnp.float32),
        jax.ShapeDtypeStruct((NW, 16), jnp.float32),
    ),
    mesh=_sc_mesh,
    compiler_params=pltpu.CompilerParams(needs_layout_passes=False),
    scratch_types=[
        pltpu.VMEM((RW, E), jnp.float32),
        pltpu.VMEM((RW, E), jnp.float32),
        pltpu.VMEM((RW // 16, 16), jnp.int32),
        pltpu.VMEM((RW // 16, 16), jnp.float32),
        pltpu.VMEM((E,), jnp.float32),
        pltpu.VMEM((E,), jnp.float32),
        pltpu.VMEM((16,), jnp.float32),
    ],
)
def _sc_stats(logits_hbm, probs_hbm, idx_hbm, tp_hbm, me_hbm, ce_hbm,
              ent_hbm, lbuf, pbuf, idxbuf, tpbuf, mebuf, cebuf, entbuf):
    wid = lax.axis_index("s") * 2 + lax.axis_index("c")
    base = wid * RW
    pltpu.sync_copy(logits_hbm.at[pl.ds(base, RW)], lbuf)

    lane0i = jnp.arange(16, dtype=jnp.int32)
    lane0 = lane0i.astype(jnp.float32)
    zeros = jnp.zeros((16,), jnp.float32)
    izeros = jnp.zeros((16,), jnp.int32)

    def row(r, carry):
        me, ce, lnacc, pdacc, idxacc, tpacc = carry
        v = [lbuf[r, pl.ds(16 * j, 16)] for j in range(4)]
        m = jnp.maximum(jnp.maximum(v[0], v[1]), jnp.maximum(v[2], v[3]))
        ms = jnp.max(m)
        d = [vj - ms for vj in v]
        e = [jnp.exp(dj) for dj in d]
        sv = (e[0] + e[1]) + (e[2] + e[3])
        sfull = zeros + jnp.sum(sv)
        rinv = 1.0 / sfull
        p = [ej * rinv for ej in e]
        for j in range(4):
            pbuf[r, pl.ds(16 * j, 16)] = p[j]
        cand = [jnp.where(d[j] >= 0.0, lane0 + (16.0 * j), 64.0)
                for j in range(4)]
        cmin = jnp.minimum(jnp.minimum(cand[0], cand[1]),
                           jnp.minimum(cand[2], cand[3]))
        msk = lane0i == (r & 15)
        idxacc = jnp.where(msk, izeros + jnp.min(cmin).astype(jnp.int32),
                           idxacc)
        tpacc = jnp.where(msk, rinv, tpacc)

        @pl.when((r & 15) == 15)
        def _flush():
            idxbuf[r >> 4, :] = idxacc
            tpbuf[r >> 4, :] = tpacc
        me = [me[j] + jnp.where(d[j] >= 0.0, 1.0, 0.0) for j in range(4)]
        ce = [ce[j] + p[j] for j in range(4)]
        bits = plsc.bitcast(sfull, jnp.int32)
        ef = (((bits >> 23) & 0xFF) - 127).astype(jnp.float32)
        mant = plsc.bitcast((bits & 0x007FFFFF) | 0x3F800000, jnp.float32)
        t = mant - 1.0
        lnm = ((((_C5 * t + _C4) * t + _C3) * t + _C2) * t + _C1) * t + _C0
        lnacc = lnacc + (_LN2 * ef + lnm) * (1.0 / 16.0)
        pdacc = pdacc + (p[0] * d[0] + p[1] * d[1]) + (p[2] * d[2]
                                                       + p[3] * d[3])
        return (me, ce, lnacc, pdacc, idxacc, tpacc)

    carry0 = ([zeros] * 4, [zeros] * 4, zeros, zeros, izeros, zeros)
    me, ce, lnacc, pdacc, _, _ = lax.fori_loop(0, RW, row, carry0)
    for j in range(4):
        mebuf[pl.ds(16 * j, 16)] = me[j]
        cebuf[pl.ds(16 * j, 16)] = ce[j]
    entbuf[...] = lnacc - pdacc

    pltpu.sync_copy(pbuf, probs_hbm.at[pl.ds(base, RW)])
    pltpu.sync_copy(idxbuf, idx_hbm.at[wid])
    pltpu.sync_copy(tpbuf, tp_hbm.at[wid])
    pltpu.sync_copy(mebuf, me_hbm.at[wid])
    pltpu.sync_copy(cebuf, ce_hbm.at[wid])
    pltpu.sync_copy(entbuf, ent_hbm.at[wid])


@functools.partial(jax.jit, static_argnames=())
def kernel(x, W, b):
    b2 = b.reshape(1, E)
    logits = _tc_logits(x, W, b2)
    probs, idx2, tp2, me_p, ce_p, ent_p = _sc_stats(logits)
    idxc = idx2.reshape(N)
    tpc = tp2.reshape(N)
    me = jnp.sum(me_p, axis=0) * (1.0 / N)
    ce = jnp.sum(ce_p, axis=0) * (1.0 / N)
    ent = jnp.sum(ent_p) * (1.0 / N)
    aux = 0.05 * (E * jnp.sum(me * ce))
    return (probs, idxc, tpc, aux, me, ce, ent)


# speedup vs baseline: 1.4851x; 1.4851x over previous
import functools

import jax
import jax.numpy as jnp
from jax.experimental import pallas as pl

N, D, E = 8192, 4096, 64
ROWS = 1024


def _router_body(x1_ref, x2_ref, w_ref, b_ref,
                 probs_ref, idx_ref, tprob_ref, aux_ref, me_ref, ce_ref,
                 ent_ref):
    i = pl.program_id(0)
    nsteps = pl.num_programs(0)

    logits = (jnp.dot(x1_ref[...], w_ref[:D // 2],
                      preferred_element_type=jnp.float32)
              + jnp.dot(x2_ref[...], w_ref[D // 2:],
                        preferred_element_type=jnp.float32)) + b_ref[...]
    m = jnp.max(logits, axis=-1, keepdims=True)
    d = logits - m
    ex = jnp.exp(d)
    s_full = jnp.dot(ex, jnp.ones((E, E), jnp.float32),
                     preferred_element_type=jnp.float32)
    rinv = 1.0 / s_full
    p = ex * rinv
    probs_ref[...] = p

    lane_f = jax.lax.broadcasted_iota(
        jnp.int32, logits.shape, 1).astype(jnp.float32)
    idx_col = jnp.min(jnp.where(d >= 0.0, lane_f, jnp.float32(E)),
                      axis=-1, keepdims=True)
    idx_ref[...] = idx_col.astype(jnp.int32)
    tprob_ref[...] = rinv[:, :1]

    one_hot = (d >= 0.0).astype(jnp.float32)
    me_part = jnp.sum(one_hot, axis=0, keepdims=True) * (1.0 / N)
    ce_part = jnp.sum(p, axis=0, keepdims=True) * (1.0 / N)
    ent_col = jnp.log(s_full[:, :1]) - jnp.sum(p * d, axis=-1,
                                               keepdims=True)
    ent_part = (jnp.sum(ent_col) * (1.0 / N)).reshape(1, 1)

    @pl.when(i == 0)
    def _init():
        me_ref[...] = me_part
        ce_ref[...] = ce_part
        ent_ref[...] = ent_part

    @pl.when(i > 0)
    def _acc():
        me_ref[...] += me_part
        ce_ref[...] += ce_part
        ent_ref[...] += ent_part

    @pl.when(i == nsteps - 1)
    def _finish():
        aux_ref[...] = 0.05 * E * jnp.sum(
            me_ref[...] * ce_ref[...]).reshape(1, 1)


@functools.partial(jax.jit, static_argnames=())
def kernel(x, W, b):
    nsteps = N // ROWS
    b2 = b.reshape(1, E)
    out_types = (
        jax.ShapeDtypeStruct((N, E), jnp.float32),
        jax.ShapeDtypeStruct((N, 1), jnp.int32),
        jax.ShapeDtypeStruct((N, 1), jnp.float32),
        jax.ShapeDtypeStruct((1, 1), jnp.float32),
        jax.ShapeDtypeStruct((1, E), jnp.float32),
        jax.ShapeDtypeStruct((1, E), jnp.float32),
        jax.ShapeDtypeStruct((1, 1), jnp.float32),
    )
    grid_spec = pl.GridSpec(
        grid=(nsteps,),
        in_specs=[
            pl.BlockSpec((ROWS, D // 2), lambda i: (i, 0)),
            pl.BlockSpec((ROWS, D // 2), lambda i: (i, 1)),
            pl.BlockSpec((D, E), lambda i: (0, 0)),
            pl.BlockSpec((1, E), lambda i: (0, 0)),
        ],
        out_specs=[
            pl.BlockSpec((ROWS, E), lambda i: (i, 0)),
            pl.BlockSpec((ROWS, 1), lambda i: (i, 0)),
            pl.BlockSpec((ROWS, 1), lambda i: (i, 0)),
            pl.BlockSpec((1, 1), lambda i: (0, 0)),
            pl.BlockSpec((1, E), lambda i: (0, 0)),
            pl.BlockSpec((1, E), lambda i: (0, 0)),
            pl.BlockSpec((1, 1), lambda i: (0, 0)),
        ],
    )
    probs, idx2, tp2, aux, me, ce, ent = pl.pallas_call(
        _router_body, grid_spec=grid_spec, out_shape=out_types)(x, x, W, b2)
    return (probs, idx2.reshape(N), tp2.reshape(N), aux[0, 0],
            me[0], ce[0], ent[0, 0])


# speedup vs baseline: 1.4945x; 1.0064x over previous
import functools

import jax
import jax.numpy as jnp
from jax.experimental import pallas as pl

N, D, E = 8192, 4096, 64
ROWS = 1024


def _router_body(x_ref, w_ref, b_ref,
                 probs_ref, idx_ref, tprob_ref, aux_ref, me_ref, ce_ref,
                 ent_ref):
    i = pl.program_id(0)
    nsteps = pl.num_programs(0)

    logits = jnp.dot(x_ref[...], w_ref[...],
                     preferred_element_type=jnp.float32) + b_ref[...]
    m = jnp.max(logits, axis=-1, keepdims=True)
    d = logits - m
    ex = jnp.exp(d)
    s_full = jnp.dot(ex, jnp.ones((E, E), jnp.float32),
                     preferred_element_type=jnp.float32)
    rinv = 1.0 / s_full
    p = ex * rinv
    probs_ref[...] = p

    lane_f = jax.lax.broadcasted_iota(
        jnp.int32, logits.shape, 1).astype(jnp.float32)
    idx_col = jnp.min(jnp.where(d >= 0.0, lane_f, jnp.float32(E)),
                      axis=-1, keepdims=True)
    idx_ref[...] = idx_col.astype(jnp.int32)
    tprob_ref[...] = rinv[:, :1]

    one_hot = (d >= 0.0).astype(jnp.float32)
    me_part = jnp.sum(one_hot, axis=0, keepdims=True) * (1.0 / N)
    ce_part = jnp.sum(p, axis=0, keepdims=True) * (1.0 / N)
    ent_col = jnp.log(s_full[:, :1]) - jnp.sum(p * d, axis=-1,
                                               keepdims=True)
    ent_part = (jnp.sum(ent_col) * (1.0 / N)).reshape(1, 1)

    @pl.when(i == 0)
    def _init():
        me_ref[...] = me_part
        ce_ref[...] = ce_part
        ent_ref[...] = ent_part

    @pl.when(i > 0)
    def _acc():
        me_ref[...] += me_part
        ce_ref[...] += ce_part
        ent_ref[...] += ent_part

    @pl.when(i == nsteps - 1)
    def _finish():
        aux_ref[...] = 0.05 * E * jnp.sum(
            me_ref[...] * ce_ref[...]).reshape(1, 1)


@functools.partial(jax.jit, static_argnames=())
def kernel(x, W, b):
    nsteps = N // ROWS
    b2 = b.reshape(1, E)
    out_types = (
        jax.ShapeDtypeStruct((N, E), jnp.float32),
        jax.ShapeDtypeStruct((N, 1), jnp.int32),
        jax.ShapeDtypeStruct((N, 1), jnp.float32),
        jax.ShapeDtypeStruct((1, 1), jnp.float32),
        jax.ShapeDtypeStruct((1, E), jnp.float32),
        jax.ShapeDtypeStruct((1, E), jnp.float32),
        jax.ShapeDtypeStruct((1, 1), jnp.float32),
    )
    grid_spec = pl.GridSpec(
        grid=(nsteps,),
        in_specs=[
            pl.BlockSpec((ROWS, D), lambda i: (i, 0)),
            pl.BlockSpec((D, E), lambda i: (0, 0)),
            pl.BlockSpec((1, E), lambda i: (0, 0)),
        ],
        out_specs=[
            pl.BlockSpec((ROWS, E), lambda i: (i, 0)),
            pl.BlockSpec((ROWS, 1), lambda i: (i, 0)),
            pl.BlockSpec((ROWS, 1), lambda i: (i, 0)),
            pl.BlockSpec((1, 1), lambda i: (0, 0)),
            pl.BlockSpec((1, E), lambda i: (0, 0)),
            pl.BlockSpec((1, E), lambda i: (0, 0)),
            pl.BlockSpec((1, 1), lambda i: (0, 0)),
        ],
    )
    probs, idx2, tp2, aux, me, ce, ent = pl.pallas_call(
        _router_body, grid_spec=grid_spec, out_shape=out_types)(x, W, b2)
    return (probs, idx2.reshape(N), tp2.reshape(N), aux[0, 0],
            me[0], ce[0], ent[0, 0])
